# Initial kernel scaffold; baseline (speedup 1.0000x reference)
import jax
import jax.numpy as jnp
from jax.experimental import pallas as pl


def kernel(chars, table):
    raise NotImplementedError("write your pallas kernel here")


# speedup vs baseline: 1.1625x; 1.1625x over previous
import functools

import jax
import jax.numpy as jnp
from jax import lax
from jax.experimental import pallas as pl
from jax.experimental.pallas import tpu as pltpu
from jax.experimental.pallas import tpu_sc as plsc

D = 128
L = 128
G = 2
NW = 32


def _build(n_rows):
    rows_per_w = n_rows // NW
    T = rows_per_w // G

    mesh = plsc.VectorSubcoreMesh(core_axis_name="c", subcore_axis_name="s")

    @functools.partial(
        pl.kernel,
        mesh=mesh,
        out_type=jax.ShapeDtypeStruct((n_rows, L, D), jnp.float32),
        scratch_types=[
            pltpu.VMEM((G, L), jnp.int32),
            pltpu.VMEM((G, L, D), jnp.float32),
            pltpu.SemaphoreType.DMA,
        ],
    )
    def gather_kernel(idx_hbm, table_hbm, out_hbm, idx_v, rows_v, sem_g):
        wid = lax.axis_index("s") * 2 + lax.axis_index("c")
        base = wid * rows_per_w

        def step(t, carry):
            r0 = base + t * G
            pltpu.sync_copy(idx_hbm.at[pl.ds(r0, G)], idx_v)
            cps = [
                pltpu.async_copy(table_hbm.at[idx_v.at[j]], rows_v.at[j], sem_g)
                for j in range(G)
            ]
            for c in cps:
                c.wait()
            pltpu.sync_copy(rows_v, out_hbm.at[pl.ds(r0, G)])
            return carry

        lax.fori_loop(0, T, step, 0)

    return gather_kernel


def kernel(chars, table):
    B, H = chars.shape
    flat = chars.reshape(-1).astype(jnp.int32)
    n = flat.shape[0]
    idx2d = flat.reshape(n // L, L)
    out = _build(n // L)(idx2d, table)
    return out.reshape(B, H, D)


# speedup vs baseline: 1.2731x; 1.0951x over previous
import functools

import jax
import jax.numpy as jnp
from jax import lax
from jax.experimental import pallas as pl
from jax.experimental.pallas import tpu as pltpu
from jax.experimental.pallas import tpu_sc as plsc

D = 128
L = 128
G = 2
NW = 32


def _build(n_rows):
    rows_per_w = n_rows // NW
    T = rows_per_w // G
    K = T // 2

    mesh = plsc.VectorSubcoreMesh(core_axis_name="c", subcore_axis_name="s")

    @functools.partial(
        pl.kernel,
        mesh=mesh,
        out_type=jax.ShapeDtypeStruct((n_rows, L, D), jnp.float32),
        scratch_types=[
            pltpu.VMEM((rows_per_w, L), jnp.int32),
            pltpu.VMEM((2, G, L, D), jnp.float32),
            pltpu.SemaphoreType.DMA,
            pltpu.SemaphoreType.DMA,
            pltpu.SemaphoreType.DMA,
            pltpu.SemaphoreType.DMA,
        ],
    )
    def gather_kernel(idx_hbm, table_hbm, out_hbm, idx_v, rows_v,
                      sg0, sg1, so0, so1):
        wid = lax.axis_index("s") * 2 + lax.axis_index("c")
        base = wid * rows_per_w
        sg = (sg0, sg1)
        so = (so0, so1)

        pltpu.sync_copy(idx_hbm.at[pl.ds(base, rows_per_w)], idx_v)

        def fire_g(t, b):
            return [
                pltpu.async_copy(table_hbm.at[idx_v.at[t * G + j]],
                                 rows_v.at[b, j], sg[b])
                for j in range(G)
            ]

        def fire_o(t, b):
            pltpu.async_copy(rows_v.at[b],
                             out_hbm.at[pl.ds(base + t * G, G)], so[b])

        def drain_o(b):
            pltpu.make_async_copy(rows_v.at[b],
                                  out_hbm.at[pl.ds(base, G)], so[b]).wait()

        def pair(t0, first):
            if not first:
                drain_o(0)
            h0 = fire_g(t0, 0)
            if not first:
                drain_o(1)
            h1 = fire_g(t0 + 1, 1)
            for h in h0:
                h.wait()
            fire_o(t0, 0)
            for h in h1:
                h.wait()
            fire_o(t0 + 1, 1)

        pair(0, True)

        def body(k, carry):
            pair(k * 2, False)
            return carry

        lax.fori_loop(1, K, body, 0)
        drain_o(0)
        drain_o(1)

    return gather_kernel


def kernel(chars, table):
    B, H = chars.shape
    flat = chars.reshape(-1).astype(jnp.int32)
    n = flat.shape[0]
    idx2d = flat.reshape(n // L, L)
    out = _build(n // L)(idx2d, table)
    return out.reshape(B, H, D)


# speedup vs baseline: 1.9631x; 1.5420x over previous
import functools

import jax
import jax.numpy as jnp
from jax import lax
from jax.experimental import pallas as pl
from jax.experimental.pallas import tpu as pltpu
from jax.experimental.pallas import tpu_sc as plsc

D = 128
NW = 32
GB = 4


def _build(B, H):
    Hp = (H + 7) // 8 * 8
    rows_per_w = B // NW
    T = rows_per_w // GB
    K = T // 2

    mesh = plsc.VectorSubcoreMesh(core_axis_name="c", subcore_axis_name="s")

    @functools.partial(
        pl.kernel,
        mesh=mesh,
        out_type=jax.ShapeDtypeStruct((B, Hp, D), jnp.float32),
        scratch_types=[
            pltpu.VMEM((rows_per_w * Hp,), jnp.int32),
            pltpu.VMEM((2, GB, Hp, D), jnp.float32),
            pltpu.SemaphoreType.DMA,
            pltpu.SemaphoreType.DMA,
            pltpu.SemaphoreType.DMA,
            pltpu.SemaphoreType.DMA,
        ],
    )
    def gather_kernel(idx_hbm, table_hbm, out_hbm, idx_v, rows_v,
                      sg0, sg1, so0, so1):
        wid = lax.axis_index("s") * 2 + lax.axis_index("c")
        base = wid * rows_per_w
        sg = (sg0, sg1)
        so = (so0, so1)

        pltpu.sync_copy(idx_hbm.at[pl.ds(base * Hp, rows_per_w * Hp)], idx_v)

        def fire_g(t, b):
            return [
                pltpu.async_copy(
                    table_hbm.at[idx_v.at[pl.ds((t * GB + j) * Hp, H)]],
                    rows_v.at[b, j, pl.ds(0, H)], sg[b])
                for j in range(GB)
            ]

        def fire_o(t, b):
            pltpu.async_copy(
                rows_v.at[b], out_hbm.at[pl.ds(base + t * GB, GB)], so[b])

        def drain_o(b):
            pltpu.make_async_copy(
                rows_v.at[b], out_hbm.at[pl.ds(base, GB)], so[b]).wait()

        def pair(t0, first):
            if not first:
                drain_o(0)
            h0 = fire_g(t0, 0)
            if not first:
                drain_o(1)
            h1 = fire_g(t0 + 1, 1)
            for h in h0:
                h.wait()
            fire_o(t0, 0)
            for h in h1:
                h.wait()
            fire_o(t0 + 1, 1)

        pair(0, True)

        def body(k, carry):
            pair(k * 2, False)
            return carry

        lax.fori_loop(1, K, body, 0)
        drain_o(0)
        drain_o(1)

    return gather_kernel


def kernel(chars, table):
    B, H = chars.shape
    Hp = (H + 7) // 8 * 8
    idx = jnp.pad(chars.astype(jnp.int32), ((0, 0), (0, Hp - H))).reshape(-1)
    out = _build(B, H)(idx, table)
    return out[:, :H, :]


# speedup vs baseline: 2.2712x; 1.1569x over previous
import functools

import jax
import jax.numpy as jnp
from jax import lax
from jax.experimental import pallas as pl
from jax.experimental.pallas import tpu as pltpu
from jax.experimental.pallas import tpu_sc as plsc

D = 128
NW = 32
GB = 4


def _build(B, H):
    Hp = (H + 7) // 8 * 8
    rows_per_w = B // NW
    T = rows_per_w // GB
    K = T // 2

    mesh = plsc.VectorSubcoreMesh(core_axis_name="c", subcore_axis_name="s")

    @functools.partial(
        pl.kernel,
        mesh=mesh,
        out_type=jax.ShapeDtypeStruct((B, H, D), jnp.float32),
        scratch_types=[
            pltpu.VMEM((rows_per_w * Hp,), jnp.int32),
            pltpu.VMEM((2, GB, H, D), jnp.float32),
            pltpu.SemaphoreType.DMA,
            pltpu.SemaphoreType.DMA,
            pltpu.SemaphoreType.DMA,
            pltpu.SemaphoreType.DMA,
        ],
    )
    def gather_kernel(idx_hbm, table_hbm, out_hbm, idx_v, rows_v,
                      sg0, sg1, so0, so1):
        wid = lax.axis_index("s") * 2 + lax.axis_index("c")
        base = wid * rows_per_w
        sg = (sg0, sg1)
        so = (so0, so1)

        pltpu.sync_copy(idx_hbm.at[pl.ds(base * Hp, rows_per_w * Hp)], idx_v)

        def fire_g(t, b):
            return [
                pltpu.async_copy(
                    table_hbm.at[idx_v.at[pl.ds((t * GB + j) * Hp, H)]],
                    rows_v.at[b, j], sg[b])
                for j in range(GB)
            ]

        def fire_o(t, b):
            pltpu.async_copy(
                rows_v.at[b], out_hbm.at[pl.ds(base + t * GB, GB)], so[b])

        def drain_o(b):
            pltpu.make_async_copy(
                rows_v.at[b], out_hbm.at[pl.ds(base, GB)], so[b]).wait()

        def pair(t0, first):
            if not first:
                drain_o(0)
            h0 = fire_g(t0, 0)
            if not first:
                drain_o(1)
            h1 = fire_g(t0 + 1, 1)
            for h in h0:
                h.wait()
            fire_o(t0, 0)
            for h in h1:
                h.wait()
            fire_o(t0 + 1, 1)

        pair(0, True)

        def body(k, carry):
            pair(k * 2, False)
            return carry

        lax.fori_loop(1, K, body, 0)
        drain_o(0)
        drain_o(1)

    return gather_kernel


def kernel(chars, table):
    B, H = chars.shape
    Hp = (H + 7) // 8 * 8
    idx = jnp.pad(chars.astype(jnp.int32), ((0, 0), (0, Hp - H))).reshape(-1)
    return _build(B, H)(idx, table)


# speedup vs baseline: 2.2737x; 1.0011x over previous
import functools

import jax
import jax.numpy as jnp
from jax import lax
from jax.experimental import pallas as pl
from jax.experimental.pallas import tpu as pltpu
from jax.experimental.pallas import tpu_sc as plsc

D = 128
NW = 32
GB = 4


def _build(B, H):
    Hp = (H + 7) // 8 * 8
    rows_per_w = B // NW
    T = rows_per_w // GB
    K = T // 2

    mesh = plsc.VectorSubcoreMesh(core_axis_name="c", subcore_axis_name="s")

    @functools.partial(
        pl.kernel,
        mesh=mesh,
        compiler_params=pltpu.CompilerParams(use_tc_tiling_on_sc=True),
        out_type=jax.ShapeDtypeStruct((B, H, D), jnp.float32),
        scratch_types=[
            pltpu.VMEM((rows_per_w * Hp,), jnp.int32),
            pltpu.VMEM((2, GB, H, D), jnp.float32),
            pltpu.SemaphoreType.DMA,
            pltpu.SemaphoreType.DMA,
            pltpu.SemaphoreType.DMA,
            pltpu.SemaphoreType.DMA,
        ],
    )
    def gather_kernel(idx_hbm, table_hbm, out_hbm, idx_v, rows_v,
                      sg0, sg1, so0, so1):
        wid = lax.axis_index("s") * 2 + lax.axis_index("c")
        base = wid * rows_per_w
        sg = (sg0, sg1)
        so = (so0, so1)

        pltpu.sync_copy(idx_hbm.at[pl.ds(base * Hp, rows_per_w * Hp)], idx_v)

        def fire_g(t, b):
            return [
                pltpu.async_copy(
                    table_hbm.at[idx_v.at[pl.ds((t * GB + j) * Hp, H)]],
                    rows_v.at[b, j], sg[b])
                for j in range(GB)
            ]

        def fire_o(t, b):
            pltpu.async_copy(
                rows_v.at[b], out_hbm.at[pl.ds(base + t * GB, GB)], so[b])

        def drain_o(b):
            pltpu.make_async_copy(
                rows_v.at[b], out_hbm.at[pl.ds(base, GB)], so[b]).wait()

        def pair(t0, first):
            if not first:
                drain_o(0)
            h0 = fire_g(t0, 0)
            if not first:
                drain_o(1)
            h1 = fire_g(t0 + 1, 1)
            for h in h0:
                h.wait()
            fire_o(t0, 0)
            for h in h1:
                h.wait()
            fire_o(t0 + 1, 1)

        pair(0, True)

        def body(k, carry):
            pair(k * 2, False)
            return carry

        lax.fori_loop(1, K, body, 0)
        drain_o(0)
        drain_o(1)

    return gather_kernel


def kernel(chars, table):
    B, H = chars.shape
    Hp = (H + 7) // 8 * 8
    idx = jnp.pad(chars.astype(jnp.int32), ((0, 0), (0, Hp - H))).reshape(-1)
    return _build(B, H)(idx, table)
